# Initial kernel scaffold; baseline (speedup 1.0000x reference)
import jax
import jax.numpy as jnp
from jax.experimental import pallas as pl


def kernel(rpn_probs, rpn_bbox, anchors):
    raise NotImplementedError("write your pallas kernel here")


# speedup vs baseline: 3.2669x; 3.2669x over previous
import functools

import jax
import jax.numpy as jnp
from jax.experimental import pallas as pl

_PROPOSALS = 1000
_NMS_T = 0.7
_PRE = 6000
_ROWS = 8
_COLS = 768


def _nms_kernel(s_ref, tid_ref, a_ref, d_ref, out_ref, *, n_prop):
    s0 = s_ref[...]
    tid = tid_ref[...]
    a0 = a_ref[0]
    a1 = a_ref[1]
    a2 = a_ref[2]
    a3 = a_ref[3]
    d0 = d_ref[0] * 0.1
    d1 = d_ref[1] * 0.1
    d2 = d_ref[2] * 0.2
    d3 = d_ref[3] * 0.2
    h = a2 - a0
    w = a3 - a1
    cy = a0 + 0.5 * h + d0 * h
    cx = a1 + 0.5 * w + d1 * w
    hh = h * jnp.exp(d2)
    ww = w * jnp.exp(d3)
    y1 = cy - 0.5 * hh
    x1 = cx - 0.5 * ww
    y2 = y1 + hh
    x2 = x1 + ww
    one = jnp.float32(1.0)
    zero = jnp.float32(0.0)
    y1 = jnp.maximum(jnp.minimum(y1, one), zero)
    x1 = jnp.maximum(jnp.minimum(x1, one), zero)
    y2 = jnp.maximum(jnp.minimum(y2, one), zero)
    x2 = jnp.maximum(jnp.minimum(x2, one), zero)
    areas = (y2 - y1) * (x2 - x1)
    lane = jax.lax.broadcasted_iota(jnp.int32, (1, 128), 1)
    m0 = (lane == 0).astype(jnp.float32)
    m1 = (lane == 1).astype(jnp.float32)
    m2 = (lane == 2).astype(jnp.float32)
    m3 = (lane == 3).astype(jnp.float32)

    def step(t, s):
        m = jnp.max(s)
        tsel = jnp.min(jnp.where(s == m, tid, jnp.int32(2147483647)))
        pm = (s == m) & (tid == tsel)
        pmf = pm.astype(jnp.float32)
        py1 = jnp.sum(pmf * y1)
        px1 = jnp.sum(pmf * x1)
        py2 = jnp.sum(pmf * y2)
        px2 = jnp.sum(pmf * x2)
        pa = jnp.sum(pmf * areas)
        valid = (m > -1e8).astype(jnp.float32)
        row = (py1 * m0 + px1 * m1 + py2 * m2 + px2 * m3) * valid
        out_ref[pl.ds(t, 1), :] = row
        yy1 = jnp.maximum(py1, y1)
        xx1 = jnp.maximum(px1, x1)
        yy2 = jnp.minimum(py2, y2)
        xx2 = jnp.minimum(px2, x2)
        inter = jnp.maximum(yy2 - yy1, zero) * jnp.maximum(xx2 - xx1, zero)
        iou = inter / (pa + areas - inter + 1e-8)
        supp = (iou > _NMS_T) | pm
        return jnp.where(supp, jnp.float32(-1e9), s)

    jax.lax.fori_loop(0, n_prop, step, s0)


def _run_nms(s_p, tid_p, a_p, d_p, n_prop, out_rows, interpret=False):
    B, R, C = s_p.shape
    f = pl.pallas_call(
        functools.partial(_nms_kernel, n_prop=n_prop),
        grid=(B,),
        in_specs=[
            pl.BlockSpec((None, R, C), lambda b: (b, 0, 0)),
            pl.BlockSpec((None, R, C), lambda b: (b, 0, 0)),
            pl.BlockSpec((None, 4, R, C), lambda b: (b, 0, 0, 0)),
            pl.BlockSpec((None, 4, R, C), lambda b: (b, 0, 0, 0)),
        ],
        out_specs=pl.BlockSpec((None, out_rows, 128), lambda b: (b, 0, 0)),
        out_shape=jax.ShapeDtypeStruct((B, out_rows, 128), jnp.float32),
        interpret=interpret,
    )
    return f(s_p, tid_p, a_p, d_p)


def kernel(rpn_probs, rpn_bbox, anchors):
    B, N, _ = rpn_probs.shape
    scores = rpn_probs[:, :, 1]
    top_s, ix = jax.lax.top_k(scores, _PRE)
    d_g = jnp.take_along_axis(rpn_bbox, ix[..., None], axis=1)
    a_g = jnp.take_along_axis(anchors, ix[..., None], axis=1)
    pad = _ROWS * _COLS - _PRE
    s_p = jnp.concatenate(
        [top_s, jnp.full((B, pad), -2e9, jnp.float32)], axis=1
    ).reshape(B, _ROWS, _COLS)
    tid_p = jnp.concatenate(
        [ix.astype(jnp.int32), jnp.full((B, pad), 2**30, jnp.int32)], axis=1
    ).reshape(B, _ROWS, _COLS)
    a_p = (
        jnp.concatenate([a_g, jnp.zeros((B, pad, 4), jnp.float32)], axis=1)
        .transpose(0, 2, 1)
        .reshape(B, 4, _ROWS, _COLS)
    )
    d_p = (
        jnp.concatenate([d_g, jnp.zeros((B, pad, 4), jnp.float32)], axis=1)
        .transpose(0, 2, 1)
        .reshape(B, 4, _ROWS, _COLS)
    )
    out = _run_nms(s_p, tid_p, a_p, d_p, _PROPOSALS, 1024)
    return out[:, :_PROPOSALS, :4]
